# Initial kernel scaffold; baseline (speedup 1.0000x reference)
import jax
import jax.numpy as jnp
from jax.experimental import pallas as pl


def kernel(h, src, dst, edge_norm, trip_src, trip_dst_i, trip_dst_j, norm_ij, norm_ik, cos_ijk, sin_ijk, mu, We0, We1, We2, be2, Wf0, Wf1, Wf2, bf2):
    raise NotImplementedError("write your pallas kernel here")


# speedup vs baseline: 3.2478x; 3.2478x over previous
import functools

import jax
import jax.numpy as jnp
from jax import lax
from jax.experimental import pallas as pl
from jax.experimental.pallas import tpu as pltpu
from jax.experimental.pallas import tpu_sc as plsc

N = 10000
E = 160000
T = 160000
F = 128
H = 64
MU = 100
INV_STEP = 10.0
MU_PAD = 1e4

NC = 2
NS = 16
LANES = 16
NW = NC * NS
C = 128
NCH = E // C
NJ = (NCH + NW - 1) // NW


def _proj_body(h_ref, w_ref, out_ref):
    out_ref[:] = lax.dot_general(
        h_ref[:], w_ref[:], (((1,), (0,)), ((), ())),
        preferred_element_type=jnp.float32)


def _project(h, wstack):
    return pl.pallas_call(
        _proj_body,
        grid=(5,),
        in_specs=[pl.BlockSpec((N, F), lambda k: (0, 0)),
                  pl.BlockSpec((F, H), lambda k: (k, 0))],
        out_specs=pl.BlockSpec((N, H), lambda k: (k, 0)),
        out_shape=jax.ShapeDtypeStruct((5 * N, H), jnp.float32),
    )(h, wstack)


def _mesh():
    return plsc.VectorSubcoreMesh(core_axis_name="c", subcore_axis_name="s")


def _gather2(table, idx_a, idx_b):
    @functools.partial(
        pl.kernel, mesh=_mesh(),
        compiler_params=pltpu.CompilerParams(use_tc_tiling_on_sc=False),
        out_type=jax.ShapeDtypeStruct((E, H), jnp.float32),
        scratch_types=[
            pltpu.VMEM((C,), jnp.int32),
            pltpu.VMEM((C,), jnp.int32),
            pltpu.VMEM((C, H), jnp.float32),
            pltpu.VMEM((C, H), jnp.float32),
            pltpu.SemaphoreType.DMA,
        ],
    )
    def k(tab_hbm, ia_hbm, ib_hbm, out_hbm, ia_v, ib_v, ra_v, rb_v, sem):
        wid = lax.axis_index("s") * NC + lax.axis_index("c")

        def step(j, _):
            ci = wid + j * NW

            @pl.when(ci < NCH)
            def _():
                base = ci * C
                pltpu.sync_copy(ia_hbm.at[pl.ds(base, C)], ia_v)
                pltpu.sync_copy(ib_hbm.at[pl.ds(base, C)], ib_v)
                ca = pltpu.async_copy(tab_hbm.at[ia_v], ra_v, sem)
                cb = pltpu.async_copy(tab_hbm.at[ib_v], rb_v, sem)
                ca.wait()
                cb.wait()

                def addrow(r, carry):
                    for q in range(H // LANES):
                        s = pl.ds(q * LANES, LANES)
                        ra_v[r, s] = ra_v[r, s] + rb_v[r, s]
                    return carry

                lax.fori_loop(0, C, addrow, 0)
                pltpu.sync_copy(ra_v, out_hbm.at[pl.ds(base, C)])

            return 0

        lax.fori_loop(0, NJ, step, 0)

    return k(table, idx_a, idx_b)


def _gather3(table, idx_a, idx_b, idx_c):
    @functools.partial(
        pl.kernel, mesh=_mesh(),
        compiler_params=pltpu.CompilerParams(use_tc_tiling_on_sc=False),
        out_type=jax.ShapeDtypeStruct((T, H), jnp.float32),
        scratch_types=[
            pltpu.VMEM((C,), jnp.int32),
            pltpu.VMEM((C,), jnp.int32),
            pltpu.VMEM((C,), jnp.int32),
            pltpu.VMEM((C, H), jnp.float32),
            pltpu.VMEM((C, H), jnp.float32),
            pltpu.VMEM((C, H), jnp.float32),
            pltpu.SemaphoreType.DMA,
        ],
    )
    def k(tab_hbm, ia_hbm, ib_hbm, ic_hbm, out_hbm,
          ia_v, ib_v, ic_v, ra_v, rb_v, rc_v, sem):
        wid = lax.axis_index("s") * NC + lax.axis_index("c")

        def step(j, _):
            ci = wid + j * NW

            @pl.when(ci < NCH)
            def _():
                base = ci * C
                pltpu.sync_copy(ia_hbm.at[pl.ds(base, C)], ia_v)
                pltpu.sync_copy(ib_hbm.at[pl.ds(base, C)], ib_v)
                pltpu.sync_copy(ic_hbm.at[pl.ds(base, C)], ic_v)
                ca = pltpu.async_copy(tab_hbm.at[ia_v], ra_v, sem)
                cb = pltpu.async_copy(tab_hbm.at[ib_v], rb_v, sem)
                cc = pltpu.async_copy(tab_hbm.at[ic_v], rc_v, sem)
                ca.wait()
                cb.wait()
                cc.wait()

                def addrow(r, carry):
                    for q in range(H // LANES):
                        s = pl.ds(q * LANES, LANES)
                        ra_v[r, s] = ra_v[r, s] + rb_v[r, s] + rc_v[r, s]
                    return carry

                lax.fori_loop(0, C, addrow, 0)
                pltpu.sync_copy(ra_v, out_hbm.at[pl.ds(base, C)])

            return 0

        lax.fori_loop(0, NJ, step, 0)

    return k(table, idx_a, idx_b, idx_c)


BLK = 2000


def _silu(x):
    return x * (1.0 / (1.0 + jnp.exp(-x)))


def _mm(a, b):
    return lax.dot_general(a, b, (((1,), (0,)), ((), ())),
                           preferred_element_type=jnp.float32)


def _edge_mlp_body(g_ref, nrm_ref, mu_ref, r_ref, w1_ref, w2_ref, b2_ref,
                   out_ref):
    d = mu_ref[:] - nrm_ref[:]
    rbf = jnp.exp(-INV_STEP * d * d)
    x = _silu(g_ref[:] + _mm(rbf, r_ref[:]))
    x = _silu(_mm(x, w1_ref[:]))
    out_ref[:] = _mm(x, w2_ref[:]) + b2_ref[:]


def _edge_mlp(g, nrm, mu_row, r_pad, w1, w2, b2, de):
    grid = E // BLK
    return pl.pallas_call(
        _edge_mlp_body,
        grid=(grid,),
        in_specs=[
            pl.BlockSpec((BLK, H), lambda i: (i, 0)),
            pl.BlockSpec((BLK, 1), lambda i: (i, 0)),
            pl.BlockSpec((1, F), lambda i: (0, 0)),
            pl.BlockSpec((F, H), lambda i: (0, 0)),
            pl.BlockSpec((H, H), lambda i: (0, 0)),
            pl.BlockSpec((H, de), lambda i: (0, 0)),
            pl.BlockSpec((1, de), lambda i: (0, 0)),
        ],
        out_specs=pl.BlockSpec((BLK, de), lambda i: (i, 0)),
        out_shape=jax.ShapeDtypeStruct((E, de), jnp.float32),
    )(g, nrm, mu_row, r_pad, w1, w2, b2)


def _trip_mlp_body(g_ref, nij_ref, nik_ref, cs_ref, mu_ref, rij_ref, rik_ref,
                   wcs_ref, w1_ref, w2_ref, b2_ref, out_ref):
    dij = mu_ref[:] - nij_ref[:]
    dik = mu_ref[:] - nik_ref[:]
    rbf_ij = jnp.exp(-INV_STEP * dij * dij)
    rbf_ik = jnp.exp(-INV_STEP * dik * dik)
    acc = g_ref[:] + _mm(rbf_ij, rij_ref[:]) + _mm(rbf_ik, rik_ref[:])
    acc = acc + _mm(cs_ref[:], wcs_ref[:])
    x = _silu(acc)
    x = _silu(_mm(x, w1_ref[:]))
    out_ref[:] = _mm(x, w2_ref[:]) + b2_ref[:]


def _trip_mlp(g, nij, nik, cs, mu_row, rij_pad, rik_pad, wcs, w1, w2, b2, dt):
    grid = T // BLK
    return pl.pallas_call(
        _trip_mlp_body,
        grid=(grid,),
        in_specs=[
            pl.BlockSpec((BLK, H), lambda i: (i, 0)),
            pl.BlockSpec((BLK, 1), lambda i: (i, 0)),
            pl.BlockSpec((BLK, 1), lambda i: (i, 0)),
            pl.BlockSpec((BLK, 2), lambda i: (i, 0)),
            pl.BlockSpec((1, F), lambda i: (0, 0)),
            pl.BlockSpec((F, H), lambda i: (0, 0)),
            pl.BlockSpec((F, H), lambda i: (0, 0)),
            pl.BlockSpec((2, H), lambda i: (0, 0)),
            pl.BlockSpec((H, H), lambda i: (0, 0)),
            pl.BlockSpec((H, dt), lambda i: (0, 0)),
            pl.BlockSpec((1, dt), lambda i: (0, 0)),
        ],
        out_specs=pl.BlockSpec((BLK, dt), lambda i: (i, 0)),
        out_shape=jax.ShapeDtypeStruct((T, dt), jnp.float32),
    )(g, nij, nik, cs, mu_row, rij_pad, rik_pad, wcs, w1, w2, b2)


def kernel(h, src, dst, edge_norm, trip_src, trip_dst_i, trip_dst_j,
           norm_ij, norm_ik, cos_ijk, sin_ijk, mu,
           We0, We1, We2, be2, Wf0, Wf1, Wf2, bf2):
    de = We2.shape[1]
    dt = Wf2.shape[1]

    wstack = jnp.concatenate(
        [We0[:F], We0[F:2 * F], Wf0[:F], Wf0[F:2 * F], Wf0[2 * F:3 * F]],
        axis=0)
    r_e = jnp.zeros((F, H), jnp.float32).at[:MU].set(We0[2 * F:])
    r_ij = jnp.zeros((F, H), jnp.float32).at[:MU].set(Wf0[3 * F:3 * F + MU])
    r_ik = jnp.zeros((F, H), jnp.float32).at[:MU].set(
        Wf0[3 * F + MU:3 * F + 2 * MU])
    wcs = Wf0[3 * F + 2 * MU:3 * F + 2 * MU + 2]
    mu_row = jnp.concatenate(
        [mu, jnp.full((F - MU,), MU_PAD, jnp.float32)]).reshape(1, F)

    table = _project(h, wstack)

    i32 = jnp.int32
    g_e = _gather2(table, src.astype(i32), (dst + N).astype(i32))
    g_t = _gather3(table, (trip_src + 2 * N).astype(i32),
                   (trip_dst_i + 3 * N).astype(i32),
                   (trip_dst_j + 4 * N).astype(i32))

    edges_weights = _edge_mlp(
        g_e, edge_norm.reshape(E, 1), mu_row, r_e, We1, We2,
        be2.reshape(1, de), de)
    cs = jnp.stack([cos_ijk, sin_ijk], axis=1)
    triplets_weights = _trip_mlp(
        g_t, norm_ij.reshape(T, 1), norm_ik.reshape(T, 1), cs, mu_row,
        r_ij, r_ik, wcs, Wf1, Wf2, bf2.reshape(1, dt), dt)
    return (edges_weights, triplets_weights)


# speedup vs baseline: 3.9539x; 1.2174x over previous
import functools

import jax
import jax.numpy as jnp
from jax import lax
from jax.experimental import pallas as pl
from jax.experimental.pallas import tpu as pltpu
from jax.experimental.pallas import tpu_sc as plsc

N = 10000
E = 160000
T = 160000
F = 128
H = 64
MU = 100
INV_STEP = 10.0
MU_PAD = 1e4

NC = 2
NS = 16
LANES = 16
NW = NC * NS
C = 128
NCH = E // C
NJ = (NCH + NW - 1) // NW


def _proj_body(h_ref, w_ref, out_ref):
    out_ref[:] = lax.dot_general(
        h_ref[:], w_ref[:], (((1,), (0,)), ((), ())),
        preferred_element_type=jnp.float32)


def _project(h, wstack):
    return pl.pallas_call(
        _proj_body,
        grid=(5,),
        in_specs=[pl.BlockSpec((N, F), lambda k: (0, 0)),
                  pl.BlockSpec((F, H), lambda k: (k, 0))],
        out_specs=pl.BlockSpec((N, H), lambda k: (k, 0)),
        out_shape=jax.ShapeDtypeStruct((5 * N, H), jnp.float32),
    )(h, wstack)


def _mesh():
    return plsc.VectorSubcoreMesh(core_axis_name="c", subcore_axis_name="s")


def _gather2(table, idx_a, idx_b):
    @functools.partial(
        pl.kernel, mesh=_mesh(),
        compiler_params=pltpu.CompilerParams(use_tc_tiling_on_sc=False),
        out_type=jax.ShapeDtypeStruct((E, H), jnp.float32),
        scratch_types=[
            pltpu.VMEM((C,), jnp.int32),
            pltpu.VMEM((C,), jnp.int32),
            pltpu.VMEM((C, H), jnp.float32),
            pltpu.VMEM((C, H), jnp.float32),
            pltpu.SemaphoreType.DMA,
        ],
    )
    def k(tab_hbm, ia_hbm, ib_hbm, out_hbm, ia_v, ib_v, ra_v, rb_v, sem):
        wid = lax.axis_index("s") * NC + lax.axis_index("c")

        def step(j, _):
            ci = wid + j * NW

            @pl.when(ci < NCH)
            def _():
                base = ci * C
                pltpu.sync_copy(ia_hbm.at[pl.ds(base, C)], ia_v)
                pltpu.sync_copy(ib_hbm.at[pl.ds(base, C)], ib_v)
                ca = pltpu.async_copy(tab_hbm.at[ia_v], ra_v, sem)
                cb = pltpu.async_copy(tab_hbm.at[ib_v], rb_v, sem)
                ca.wait()
                cb.wait()

                def addrow(r, carry):
                    for q in range(H // LANES):
                        s = pl.ds(q * LANES, LANES)
                        ra_v[r, s] = ra_v[r, s] + rb_v[r, s]
                    return carry

                lax.fori_loop(0, C, addrow, 0)
                pltpu.sync_copy(ra_v, out_hbm.at[pl.ds(base, C)])

            return 0

        lax.fori_loop(0, NJ, step, 0)

    return k(table, idx_a, idx_b)


def _gather3(table, idx_a, idx_b, idx_c):
    @functools.partial(
        pl.kernel, mesh=_mesh(),
        compiler_params=pltpu.CompilerParams(use_tc_tiling_on_sc=False),
        out_type=jax.ShapeDtypeStruct((T, H), jnp.float32),
        scratch_types=[
            pltpu.VMEM((C,), jnp.int32),
            pltpu.VMEM((C,), jnp.int32),
            pltpu.VMEM((C,), jnp.int32),
            pltpu.VMEM((C, H), jnp.float32),
            pltpu.VMEM((C, H), jnp.float32),
            pltpu.VMEM((C, H), jnp.float32),
            pltpu.SemaphoreType.DMA,
        ],
    )
    def k(tab_hbm, ia_hbm, ib_hbm, ic_hbm, out_hbm,
          ia_v, ib_v, ic_v, ra_v, rb_v, rc_v, sem):
        wid = lax.axis_index("s") * NC + lax.axis_index("c")

        def step(j, _):
            ci = wid + j * NW

            @pl.when(ci < NCH)
            def _():
                base = ci * C
                pltpu.sync_copy(ia_hbm.at[pl.ds(base, C)], ia_v)
                pltpu.sync_copy(ib_hbm.at[pl.ds(base, C)], ib_v)
                pltpu.sync_copy(ic_hbm.at[pl.ds(base, C)], ic_v)
                ca = pltpu.async_copy(tab_hbm.at[ia_v], ra_v, sem)
                cb = pltpu.async_copy(tab_hbm.at[ib_v], rb_v, sem)
                cc = pltpu.async_copy(tab_hbm.at[ic_v], rc_v, sem)
                ca.wait()
                cb.wait()
                cc.wait()

                def addrow(r, carry):
                    for q in range(H // LANES):
                        s = pl.ds(q * LANES, LANES)
                        ra_v[r, s] = ra_v[r, s] + rb_v[r, s] + rc_v[r, s]
                    return carry

                lax.fori_loop(0, C, addrow, 0)
                pltpu.sync_copy(ra_v, out_hbm.at[pl.ds(base, C)])

            return 0

        lax.fori_loop(0, NJ, step, 0)

    return k(table, idx_a, idx_b, idx_c)


BLK = 1280
CH = BLK // F


def _silu(x):
    return x * (1.0 / (1.0 + jnp.exp(-x)))


def _mm(a, b):
    return lax.dot_general(a, b, (((1,), (0,)), ((), ())),
                           preferred_element_type=jnp.float32)


def _mm_t(a, b):
    return lax.dot_general(a, b, (((0,), (0,)), ((), ())),
                           preferred_element_type=jnp.float32)


def _edge_mlp_body(g_ref, nrm_ref, mu_ref, r_ref, w1_ref, w2_ref, b2_ref,
                   out_ref):
    mu_c = mu_ref[:]
    chunks = []
    for k in range(CH):
        d = mu_c - nrm_ref[0, k:k + 1, :]
        rbf = jnp.exp(-INV_STEP * d * d)
        chunks.append(_mm_t(rbf, r_ref[:]))
    x = _silu(g_ref[:] + jnp.concatenate(chunks, axis=0))
    x = _silu(_mm(x, w1_ref[:]))
    out_ref[:] = _mm(x, w2_ref[:]) + b2_ref[:]


def _edge_mlp(g, nrm2, mu_col, r_pad, w1, w2, b2, de):
    grid = E // BLK
    return pl.pallas_call(
        _edge_mlp_body,
        grid=(grid,),
        in_specs=[
            pl.BlockSpec((BLK, H), lambda i: (i, 0)),
            pl.BlockSpec((1, CH, F), lambda i: (i, 0, 0)),
            pl.BlockSpec((F, 1), lambda i: (0, 0)),
            pl.BlockSpec((F, H), lambda i: (0, 0)),
            pl.BlockSpec((H, H), lambda i: (0, 0)),
            pl.BlockSpec((H, de), lambda i: (0, 0)),
            pl.BlockSpec((1, de), lambda i: (0, 0)),
        ],
        out_specs=pl.BlockSpec((BLK, de), lambda i: (i, 0)),
        out_shape=jax.ShapeDtypeStruct((E, de), jnp.float32),
    )(g, nrm2, mu_col, r_pad, w1, w2, b2)


def _trip_mlp_body(g_ref, nij_ref, nik_ref, cos_ref, sin_ref, mu_ref,
                   rij_ref, rik_ref, wcs_ref, w1_ref, w2_ref, b2_ref,
                   out_ref):
    mu_c = mu_ref[:]
    chunks = []
    for k in range(CH):
        dij = mu_c - nij_ref[0, k:k + 1, :]
        dik = mu_c - nik_ref[0, k:k + 1, :]
        rbf_ij = jnp.exp(-INV_STEP * dij * dij)
        rbf_ik = jnp.exp(-INV_STEP * dik * dik)
        cs = jnp.concatenate([cos_ref[0, k:k + 1, :], sin_ref[0, k:k + 1, :]],
                             axis=0)
        c = (_mm_t(rbf_ij, rij_ref[:]) + _mm_t(rbf_ik, rik_ref[:])
             + _mm_t(cs, wcs_ref[:]))
        chunks.append(c)
    x = _silu(g_ref[:] + jnp.concatenate(chunks, axis=0))
    x = _silu(_mm(x, w1_ref[:]))
    out_ref[:] = _mm(x, w2_ref[:]) + b2_ref[:]


def _trip_mlp(g, nij2, nik2, cos2, sin2, mu_col, rij_pad, rik_pad, wcs,
              w1, w2, b2, dt):
    grid = T // BLK
    return pl.pallas_call(
        _trip_mlp_body,
        grid=(grid,),
        in_specs=[
            pl.BlockSpec((BLK, H), lambda i: (i, 0)),
            pl.BlockSpec((1, CH, F), lambda i: (i, 0, 0)),
            pl.BlockSpec((1, CH, F), lambda i: (i, 0, 0)),
            pl.BlockSpec((1, CH, F), lambda i: (i, 0, 0)),
            pl.BlockSpec((1, CH, F), lambda i: (i, 0, 0)),
            pl.BlockSpec((F, 1), lambda i: (0, 0)),
            pl.BlockSpec((F, H), lambda i: (0, 0)),
            pl.BlockSpec((F, H), lambda i: (0, 0)),
            pl.BlockSpec((2, H), lambda i: (0, 0)),
            pl.BlockSpec((H, H), lambda i: (0, 0)),
            pl.BlockSpec((H, dt), lambda i: (0, 0)),
            pl.BlockSpec((1, dt), lambda i: (0, 0)),
        ],
        out_specs=pl.BlockSpec((BLK, dt), lambda i: (i, 0)),
        out_shape=jax.ShapeDtypeStruct((T, dt), jnp.float32),
    )(g, nij2, nik2, cos2, sin2, mu_col, rij_pad, rik_pad, wcs, w1, w2, b2)


def kernel(h, src, dst, edge_norm, trip_src, trip_dst_i, trip_dst_j,
           norm_ij, norm_ik, cos_ijk, sin_ijk, mu,
           We0, We1, We2, be2, Wf0, Wf1, Wf2, bf2):
    de = We2.shape[1]
    dt = Wf2.shape[1]

    wstack = jnp.concatenate(
        [We0[:F], We0[F:2 * F], Wf0[:F], Wf0[F:2 * F], Wf0[2 * F:3 * F]],
        axis=0)
    r_e = jnp.zeros((F, H), jnp.float32).at[:MU].set(We0[2 * F:])
    r_ij = jnp.zeros((F, H), jnp.float32).at[:MU].set(Wf0[3 * F:3 * F + MU])
    r_ik = jnp.zeros((F, H), jnp.float32).at[:MU].set(
        Wf0[3 * F + MU:3 * F + 2 * MU])
    wcs = Wf0[3 * F + 2 * MU:3 * F + 2 * MU + 2]
    mu_col = jnp.concatenate(
        [mu, jnp.full((F - MU,), MU_PAD, jnp.float32)]).reshape(F, 1)

    table = _project(h, wstack)

    i32 = jnp.int32
    g_e = _gather2(table, src.astype(i32), (dst + N).astype(i32))
    g_t = _gather3(table, (trip_src + 2 * N).astype(i32),
                   (trip_dst_i + 3 * N).astype(i32),
                   (trip_dst_j + 4 * N).astype(i32))

    edges_weights = _edge_mlp(
        g_e, edge_norm.reshape(E // BLK, CH, F), mu_col, r_e, We1, We2,
        be2.reshape(1, de), de)
    triplets_weights = _trip_mlp(
        g_t, norm_ij.reshape(T // BLK, CH, F), norm_ik.reshape(T // BLK, CH, F),
        cos_ijk.reshape(T // BLK, CH, F), sin_ijk.reshape(T // BLK, CH, F),
        mu_col,
        r_ij, r_ik, wcs, Wf1, Wf2, bf2.reshape(1, dt), dt)
    return (edges_weights, triplets_weights)
